# baseline (speedup 1.0000x reference)
import functools
import math

import jax
import jax.numpy as jnp
from jax import lax
from jax.experimental import pallas as pl
from jax.experimental.pallas import tpu as pltpu
from jax.experimental.pallas import tpu_sc as plsc

D_MODEL = 128
SCALE = math.sqrt(float(D_MODEL))

_NC = 2
_NS = 16
_NW = _NC * _NS
_L = 16

NBUF = 4


def _make_gather(B, S, D):
    assert B % _NW == 0
    rows_per_w = B // _NW
    n_groups = rows_per_w // NBUF
    assert rows_per_w % NBUF == 0 and n_groups >= 2

    mesh = plsc.VectorSubcoreMesh(core_axis_name="c", subcore_axis_name="s")

    @functools.partial(
        pl.kernel,
        mesh=mesh,
        out_type=jax.ShapeDtypeStruct((B, S, D), jnp.float32),
        compiler_params=pltpu.CompilerParams(use_tc_tiling_on_sc=True),
        scratch_types=[
            pltpu.VMEM((rows_per_w, S), jnp.int32),
            *([pltpu.VMEM((S, D), jnp.float32)] * NBUF),
            *([pltpu.VMEM((S, D), jnp.float32)] * NBUF),
            *([pltpu.SemaphoreType.DMA] * NBUF),
            *([pltpu.SemaphoreType.DMA] * NBUF),
        ],
    )
    def gather_kernel(table_hbm, idx_hbm, out_hbm, idx_v, *bufs):
        gbuf = bufs[:NBUF]
        obuf = bufs[NBUF:2 * NBUF]
        gsem = bufs[2 * NBUF:3 * NBUF]
        ssem = bufs[3 * NBUF:4 * NBUF]

        wid = lax.axis_index("s") * _NC + lax.axis_index("c")
        base = wid * rows_per_w
        pltpu.sync_copy(idx_hbm.at[wid], idx_v)

        def issue_gather(r, b):
            pltpu.make_async_copy(
                table_hbm.at[idx_v.at[r]], gbuf[b], gsem[b]).start()

        def issue_scatter(r, b):
            pltpu.make_async_copy(
                obuf[b], out_hbm.at[base + r], ssem[b]).start()

        def wait_gather(b):
            pltpu.make_async_copy(
                table_hbm.at[idx_v.at[0]], gbuf[b], gsem[b]).wait()

        def wait_scatter(b):
            pltpu.make_async_copy(
                obuf[b], out_hbm.at[base], ssem[b]).wait()

        def multiply(b):
            def row_body(r, carry):
                for j in range(D // _L):
                    sl = pl.ds(j * _L, _L)
                    obuf[b][r, sl] = gbuf[b][r, sl] * SCALE
                return carry

            lax.fori_loop(0, S, row_body, 0)

        for b in range(NBUF):
            issue_gather(b, b)
        for b in range(NBUF):
            wait_gather(b)
            multiply(b)
            issue_gather(NBUF + b, b)
            issue_scatter(b, b)

        def group_body(g, carry):
            for b in range(NBUF):
                r = g * NBUF + b
                wait_gather(b)
                wait_scatter(b)
                multiply(b)
                issue_gather(r + NBUF, b)
                issue_scatter(r, b)
            return carry

        lax.fori_loop(1, n_groups - 1, group_body, 0)

        for b in range(NBUF):
            r = (n_groups - 1) * NBUF + b
            wait_gather(b)
            wait_scatter(b)
            multiply(b)
            issue_scatter(r, b)

        for b in range(NBUF):
            wait_scatter(b)

    return gather_kernel


def kernel(x, target_vec, table, W, b):
    B, S = x.shape
    V, D = table.shape
    rows_per_w = B // _NW
    idx = x.reshape(_NW, rows_per_w, S).astype(jnp.int32)
    return _make_gather(B, S, D)(table, idx)


# speedup vs baseline: 1.0043x; 1.0043x over previous
import functools
import math

import jax
import jax.numpy as jnp
from jax import lax
from jax.experimental import pallas as pl
from jax.experimental.pallas import tpu as pltpu
from jax.experimental.pallas import tpu_sc as plsc

D_MODEL = 128
SCALE = math.sqrt(float(D_MODEL))

_NC = 2
_NS = 16
_NW = _NC * _NS
_L = 16

NBUF = 4


def _make_gather(B, S, D):
    assert B % _NW == 0
    rows_per_w = B // _NW
    n_groups = rows_per_w // NBUF
    assert rows_per_w % NBUF == 0 and n_groups >= 2

    mesh = plsc.VectorSubcoreMesh(core_axis_name="c", subcore_axis_name="s")

    @functools.partial(
        pl.kernel,
        mesh=mesh,
        out_type=jax.ShapeDtypeStruct((B, S, D), jnp.float32),
        compiler_params=pltpu.CompilerParams(
            use_tc_tiling_on_sc=True, needs_layout_passes=True),
        scratch_types=[
            pltpu.VMEM((rows_per_w, S), jnp.int32),
            *([pltpu.VMEM((S, D), jnp.float32)] * NBUF),
            *([pltpu.VMEM((S, D), jnp.float32)] * NBUF),
            *([pltpu.SemaphoreType.DMA] * NBUF),
            *([pltpu.SemaphoreType.DMA] * NBUF),
        ],
    )
    def gather_kernel(table_hbm, idx_hbm, out_hbm, idx_v, *bufs):
        gbuf = bufs[:NBUF]
        obuf = bufs[NBUF:2 * NBUF]
        gsem = bufs[2 * NBUF:3 * NBUF]
        ssem = bufs[3 * NBUF:4 * NBUF]

        wid = lax.axis_index("s") * _NC + lax.axis_index("c")
        base = wid * rows_per_w
        pltpu.sync_copy(idx_hbm.at[wid], idx_v)

        def issue_gather(r, b):
            pltpu.make_async_copy(
                table_hbm.at[idx_v.at[r]], gbuf[b], gsem[b]).start()

        def issue_scatter(r, b):
            pltpu.make_async_copy(
                obuf[b], out_hbm.at[base + r], ssem[b]).start()

        def wait_gather(b):
            pltpu.make_async_copy(
                table_hbm.at[idx_v.at[0]], gbuf[b], gsem[b]).wait()

        def wait_scatter(b):
            pltpu.make_async_copy(
                obuf[b], out_hbm.at[base], ssem[b]).wait()

        def multiply(b):
            def row_body(r, carry):
                for j in range(D // _L):
                    sl = pl.ds(j * _L, _L)
                    obuf[b][r, sl] = gbuf[b][r, sl] * SCALE
                return carry

            lax.fori_loop(0, S, row_body, 0)

        for b in range(NBUF):
            issue_gather(b, b)
        for b in range(NBUF):
            wait_gather(b)
            multiply(b)
            issue_gather(NBUF + b, b)
            issue_scatter(b, b)

        def group_body(g, carry):
            for b in range(NBUF):
                r = g * NBUF + b
                wait_gather(b)
                wait_scatter(b)
                multiply(b)
                issue_gather(r + NBUF, b)
                issue_scatter(r, b)
            return carry

        lax.fori_loop(1, n_groups - 1, group_body, 0)

        for b in range(NBUF):
            r = (n_groups - 1) * NBUF + b
            wait_gather(b)
            wait_scatter(b)
            multiply(b)
            issue_scatter(r, b)

        for b in range(NBUF):
            wait_scatter(b)

    return gather_kernel


def kernel(x, target_vec, table, W, b):
    B, S = x.shape
    V, D = table.shape
    rows_per_w = B // _NW
    idx = x.reshape(_NW, rows_per_w, S).astype(jnp.int32)
    return _make_gather(B, S, D)(table, idx)
